# baseline (speedup 1.0000x reference)
import functools

import jax
import jax.numpy as jnp
from jax.experimental import pallas as pl
from jax.experimental.pallas import tpu as pltpu
from jax.experimental.pallas import tpu_sc as plsc

_EMBED = 64
_L_OUT = 200
_WINDOW = 128


def _sc_gather(table, flat_idx):
    n = flat_idx.shape[0]
    idx2 = flat_idx.reshape(1, n)
    mesh = plsc.VectorSubcoreMesh(core_axis_name="core", subcore_axis_name="subcore")

    @functools.partial(
        pl.kernel,
        out_type=jax.ShapeDtypeStruct((n, _EMBED), table.dtype),
        mesh=mesh,
        compiler_params=pltpu.CompilerParams(use_tc_tiling_on_sc=False),
    )
    def gather_kernel(x_hbm, i_hbm, o_hbm):
        def body(i_vmem, o_vmem):
            pltpu.sync_copy(x_hbm.at[i_vmem.at[0]], o_vmem)

        pltpu.emit_pipeline(
            body,
            grid=(n // _WINDOW,),
            in_specs=[pl.BlockSpec((1, _WINDOW), index_map=lambda i: (0, i))],
            out_specs=[pl.BlockSpec((_WINDOW, _EMBED), index_map=lambda i: (i, 0))],
            core_axis_name=("core", "subcore"),
            dimension_semantics=(pltpu.PARALLEL,),
        )(i_hbm, o_hbm)

    return gather_kernel(table, idx2)


def _tc_transpose_pad(interm, l_in, block_b=8):
    b = interm.shape[0]

    def body(x_ref, o_ref):
        t = jnp.swapaxes(x_ref[...], 1, 2)
        if l_in < _L_OUT:
            pad = jnp.zeros((block_b, _EMBED, _L_OUT - l_in), t.dtype)
            t = jnp.concatenate([t, pad], axis=2)
        o_ref[...] = t

    return pl.pallas_call(
        body,
        grid=(b // block_b,),
        in_specs=[pl.BlockSpec((block_b, l_in, _EMBED), lambda i: (i, 0, 0))],
        out_specs=pl.BlockSpec((block_b, _EMBED, _L_OUT), lambda i: (i, 0, 0)),
        out_shape=jax.ShapeDtypeStruct((b, _EMBED, _L_OUT), interm.dtype),
    )(interm)


def kernel(query_input, document_input, embedding_table):
    bq, lq = query_input.shape
    bd, ld = document_input.shape

    q_rows = _sc_gather(embedding_table, query_input.reshape(-1))
    d_rows = _sc_gather(embedding_table, document_input.reshape(-1))

    q_out = _tc_transpose_pad(q_rows.reshape(bq, lq, _EMBED), lq)
    d_out = _tc_transpose_pad(d_rows.reshape(bd, ld, _EMBED), ld)
    return q_out, d_out


# speedup vs baseline: 1.0833x; 1.0833x over previous
import functools

import jax
import jax.numpy as jnp
from jax import lax
from jax.experimental import pallas as pl
from jax.experimental.pallas import tpu as pltpu
from jax.experimental.pallas import tpu_sc as plsc

_EMBED = 64
_PAIR = 128
_L_OUT = 200
_NUM_WORKERS = 32


def _sc_gather_pairs(table2, flat_idx, window):
    n = flat_idx.shape[0]
    nw = n // window
    wpw = nw // _NUM_WORKERS
    assert nw * window == n and wpw * _NUM_WORKERS == nw and wpw % 2 == 0
    idx2 = flat_idx.reshape(_NUM_WORKERS, wpw, window)
    mesh = plsc.VectorSubcoreMesh(core_axis_name="core", subcore_axis_name="subcore")

    @functools.partial(
        pl.kernel,
        out_type=jax.ShapeDtypeStruct((n, _PAIR), table2.dtype),
        mesh=mesh,
        scratch_types=[
            pltpu.VMEM((wpw, window), jnp.int32),
            pltpu.VMEM((window, _PAIR), jnp.float32),
            pltpu.VMEM((window, _PAIR), jnp.float32),
            pltpu.SemaphoreType.DMA,
            pltpu.SemaphoreType.DMA,
            pltpu.SemaphoreType.DMA,
            pltpu.SemaphoreType.DMA,
        ],
    )
    def gather_kernel(tab_hbm, idx_hbm, out_hbm, idx_v, buf_a, buf_b, gsem_a,
                      gsem_b, wsem_a, wsem_b):
        wid = lax.axis_index("subcore") * 2 + lax.axis_index("core")
        base = wid * wpw

        pltpu.sync_copy(idx_hbm.at[wid], idx_v)

        @pl.loop(0, wpw)
        def _(w):
            for j in range(window // 16):
                sl = pl.ds(j * 16, 16)
                idx_v[w, sl] = lax.shift_right_logical(idx_v[w, sl], 1)

        def g_start(w, buf, sem):
            pltpu.async_copy(tab_hbm.at[idx_v.at[w]], buf, sem)

        def g_wait(w, buf, sem):
            pltpu.make_async_copy(tab_hbm.at[idx_v.at[w]], buf, sem).wait()

        def out_at(w):
            off = pl.multiple_of((base + w) * window, window)
            return out_hbm.at[pl.ds(off, window)]

        def wb_start(w, buf, sem):
            pltpu.async_copy(buf, out_at(w), sem)

        def wb_wait(w, buf, sem):
            pltpu.make_async_copy(buf, out_at(w), sem).wait()

        npairs = wpw // 2
        g_start(0, buf_a, gsem_a)

        @pl.loop(0, npairs)
        def _(i):
            w0 = 2 * i
            g_start(w0 + 1, buf_b, gsem_b)
            g_wait(w0, buf_a, gsem_a)
            wb_start(w0, buf_a, wsem_a)
            g_wait(w0 + 1, buf_b, gsem_b)
            wb_start(w0 + 1, buf_b, wsem_b)

            @pl.when(i < npairs - 1)
            def _():
                wb_wait(w0, buf_a, wsem_a)
                g_start(w0 + 2, buf_a, gsem_a)
                wb_wait(w0 + 1, buf_b, wsem_b)

        wb_wait(wpw - 2, buf_a, wsem_a)
        wb_wait(wpw - 1, buf_b, wsem_b)

    return gather_kernel(table2, idx2)


def _tc_select_transpose_pad(pairs, idx, block_b=8):
    b, l_in = idx.shape

    def body(x_ref, i_ref, o_ref):
        t = jnp.swapaxes(x_ref[...], 1, 2)
        par = i_ref[...] & 1
        sel = jnp.where((par == 0)[:, None, :], t[:, :_EMBED, :],
                        t[:, _EMBED:, :])
        if l_in < _L_OUT:
            pad = jnp.zeros((block_b, _EMBED, _L_OUT - l_in), sel.dtype)
            sel = jnp.concatenate([sel, pad], axis=2)
        o_ref[...] = sel

    return pl.pallas_call(
        body,
        grid=(b // block_b,),
        in_specs=[
            pl.BlockSpec((block_b, l_in, _PAIR), lambda i: (i, 0, 0)),
            pl.BlockSpec((block_b, l_in), lambda i: (i, 0)),
        ],
        out_specs=pl.BlockSpec((block_b, _EMBED, _L_OUT), lambda i: (i, 0, 0)),
        out_shape=jax.ShapeDtypeStruct((b, _EMBED, _L_OUT), pairs.dtype),
    )(pairs, idx)


def kernel(query_input, document_input, embedding_table):
    bq, lq = query_input.shape
    bd, ld = document_input.shape
    table2 = embedding_table.reshape(-1, _PAIR)

    q_pairs = _sc_gather_pairs(table2, query_input.reshape(-1), window=64)
    d_pairs = _sc_gather_pairs(table2, document_input.reshape(-1), window=128)

    q_out = _tc_select_transpose_pad(q_pairs.reshape(bq, lq, _PAIR), query_input)
    d_out = _tc_select_transpose_pad(d_pairs.reshape(bd, ld, _PAIR), document_input)
    return q_out, d_out


# speedup vs baseline: 1.5641x; 1.4439x over previous
import functools

import jax
import jax.numpy as jnp
from jax import lax
from jax.experimental import pallas as pl
from jax.experimental.pallas import tpu as pltpu
from jax.experimental.pallas import tpu_sc as plsc

_EMBED = 64
_ROW = 128
_L_OUT = 200
_NUM_WORKERS = 32


def _tc_format_table(table_t, lane_block=2048, chunk=512):
    v = table_t.shape[1]

    def body(x_ref, o_ref):
        for k in range(lane_block // chunk):
            rows = pl.ds(k * chunk, chunk)
            o_ref[rows, 0:_EMBED] = x_ref[:, k * chunk:(k + 1) * chunk].T

    return pl.pallas_call(
        body,
        grid=(pl.cdiv(v, lane_block),),
        in_specs=[pl.BlockSpec((_EMBED, lane_block), lambda i: (0, i))],
        out_specs=pl.BlockSpec((lane_block, _ROW), lambda i: (i, 0)),
        out_shape=jax.ShapeDtypeStruct((v, _ROW), table_t.dtype),
    )(table_t)


def _sc_gather_rows(table2, flat_idx, window):
    n = flat_idx.shape[0]
    nw = n // window
    wpw = nw // _NUM_WORKERS
    assert nw * window == n and wpw * _NUM_WORKERS == nw and wpw % 2 == 0
    idx2 = flat_idx.reshape(_NUM_WORKERS, wpw, window)
    mesh = plsc.VectorSubcoreMesh(core_axis_name="core", subcore_axis_name="subcore")

    @functools.partial(
        pl.kernel,
        out_type=jax.ShapeDtypeStruct((n, _ROW), table2.dtype),
        mesh=mesh,
        scratch_types=[
            pltpu.VMEM((wpw, window), jnp.int32),
            pltpu.VMEM((window, _ROW), jnp.float32),
            pltpu.VMEM((window, _ROW), jnp.float32),
            pltpu.SemaphoreType.DMA,
            pltpu.SemaphoreType.DMA,
            pltpu.SemaphoreType.DMA,
            pltpu.SemaphoreType.DMA,
        ],
    )
    def gather_kernel(tab_hbm, idx_hbm, out_hbm, idx_v, buf_a, buf_b, gsem_a,
                      gsem_b, wsem_a, wsem_b):
        wid = lax.axis_index("subcore") * 2 + lax.axis_index("core")
        base = wid * wpw

        pltpu.sync_copy(idx_hbm.at[wid], idx_v)

        def g_start(w, buf, sem):
            pltpu.async_copy(tab_hbm.at[idx_v.at[w]], buf, sem)

        def g_wait(w, buf, sem):
            pltpu.make_async_copy(tab_hbm.at[idx_v.at[w]], buf, sem).wait()

        def out_at(w):
            off = pl.multiple_of((base + w) * window, window)
            return out_hbm.at[pl.ds(off, window)]

        def wb_start(w, buf, sem):
            pltpu.async_copy(buf, out_at(w), sem)

        def wb_wait(w, buf, sem):
            pltpu.make_async_copy(buf, out_at(w), sem).wait()

        npairs = wpw // 2
        g_start(0, buf_a, gsem_a)

        @pl.loop(0, npairs)
        def _(i):
            w0 = 2 * i
            g_start(w0 + 1, buf_b, gsem_b)
            g_wait(w0, buf_a, gsem_a)
            wb_start(w0, buf_a, wsem_a)
            g_wait(w0 + 1, buf_b, gsem_b)
            wb_start(w0 + 1, buf_b, wsem_b)

            @pl.when(i < npairs - 1)
            def _():
                wb_wait(w0, buf_a, wsem_a)
                g_start(w0 + 2, buf_a, gsem_a)
                wb_wait(w0 + 1, buf_b, wsem_b)

        wb_wait(wpw - 2, buf_a, wsem_a)
        wb_wait(wpw - 1, buf_b, wsem_b)

    return gather_kernel(table2, idx2)


def _tc_transpose_t(rows3, block_b=128, block_l=40):
    b, l_in, _ = rows3.shape

    if l_in == _L_OUT:
        def body(x_ref, o_ref):
            t = jnp.transpose(x_ref[...], (2, 1, 0))
            o_ref[...] = t[:_EMBED]

        return pl.pallas_call(
            body,
            grid=(b // block_b, l_in // block_l),
            in_specs=[
                pl.BlockSpec((block_b, block_l, _ROW), lambda i, j: (i, j, 0)),
            ],
            out_specs=pl.BlockSpec((_EMBED, block_l, block_b),
                                   lambda i, j: (0, j, i)),
            out_shape=jax.ShapeDtypeStruct((_EMBED, _L_OUT, b), rows3.dtype),
        )(rows3)

    def body(x_ref, o_ref):
        t = jnp.transpose(x_ref[...], (2, 1, 0))[:_EMBED]
        pad = jnp.zeros((_EMBED, _L_OUT - l_in, block_b), t.dtype)
        o_ref[...] = jnp.concatenate([t, pad], axis=1)

    return pl.pallas_call(
        body,
        grid=(b // block_b,),
        in_specs=[
            pl.BlockSpec((block_b, l_in, _ROW), lambda i: (i, 0, 0)),
        ],
        out_specs=pl.BlockSpec((_EMBED, _L_OUT, block_b), lambda i: (0, 0, i)),
        out_shape=jax.ShapeDtypeStruct((_EMBED, _L_OUT, b), rows3.dtype),
    )(rows3)


def kernel(query_input, document_input, embedding_table):
    bq, lq = query_input.shape
    bd, ld = document_input.shape
    table_t = jnp.swapaxes(embedding_table, 0, 1)
    table2 = _tc_format_table(table_t)

    q_rows = _sc_gather_rows(table2, query_input.reshape(-1), window=64)
    d_rows = _sc_gather_rows(table2, document_input.reshape(-1), window=128)

    q_t = _tc_transpose_t(q_rows.reshape(bq, lq, _ROW))
    d_t = _tc_transpose_t(d_rows.reshape(bd, ld, _ROW))
    return jnp.transpose(q_t, (2, 0, 1)), jnp.transpose(d_t, (2, 0, 1))
